# Initial kernel scaffold; baseline (speedup 1.0000x reference)
import jax
import jax.numpy as jnp
from jax.experimental import pallas as pl


def kernel(x):
    raise NotImplementedError("write your pallas kernel here")


# speedup vs baseline: 2.6665x; 2.6665x over previous
import functools

import jax
import jax.numpy as jnp
from jax import lax
from jax.experimental import pallas as pl
from jax.experimental.pallas import tpu as pltpu
from jax.experimental.pallas import tpu_sc as plsc

TOTAL_TOKENS = 1024
RETAIN = 256
BATCH = 64
C = 768

ROWS = BATCH * TOTAL_TOKENS
ROWS_Y = BATCH * RETAIN
NW = 32
RPW = ROWS // NW
Y_WORKERS = ROWS_Y // RPW
CHUNK = 64
NCH = RPW // CHUNK


def _build_sampler_kernel():
    info = plsc.get_sparse_core_info()
    nc = info.num_cores
    mesh = plsc.VectorSubcoreMesh(core_axis_name="c", subcore_axis_name="s")

    @functools.partial(
        pl.kernel,
        mesh=mesh,
        out_type=(
            jax.ShapeDtypeStruct((ROWS_Y, C), jnp.float32),
            jax.ShapeDtypeStruct((ROWS - ROWS_Y, C), jnp.float32),
        ),
        scratch_types=[
            pltpu.VMEM((NCH, CHUNK), jnp.int32),
            pltpu.VMEM((CHUNK, C), jnp.float32),
            pltpu.SemaphoreType.DMA,
        ],
    )
    def sampler(x_hbm, idx_hbm, y_hbm, z_hbm, idx_v, buf, gsem):
        w = lax.axis_index("s") * nc + lax.axis_index("c")
        pltpu.sync_copy(idx_hbm.at[w], idx_v)

        def run(out_ref, obase):
            def body(c, carry):
                pltpu.async_copy(x_hbm.at[idx_v.at[c]], buf, gsem).wait()
                pltpu.sync_copy(buf, out_ref.at[pl.ds(obase + c * CHUNK, CHUNK)])
                return carry

            lax.fori_loop(0, NCH, body, 0)

        @pl.when(w < Y_WORKERS)
        def _():
            run(y_hbm, w * RPW)

        @pl.when(w >= Y_WORKERS)
        def _():
            run(z_hbm, (w - Y_WORKERS) * RPW)

    return sampler


_sampler = _build_sampler_kernel()


def kernel(x):
    perm = jax.random.permutation(jax.random.key(1), TOTAL_TOKENS)
    row_base = (jnp.arange(BATCH, dtype=jnp.int32) * TOTAL_TOKENS)[:, None]
    idx_y = (row_base + perm[None, :RETAIN]).reshape(-1)
    idx_z = (row_base + perm[None, RETAIN:]).reshape(-1)
    idx = (
        jnp.concatenate([idx_y, idx_z])
        .astype(jnp.int32)
        .reshape(NW, NCH, CHUNK)
    )
    y_flat, z_flat = _sampler(x.reshape(ROWS, C), idx)
    return (
        y_flat.reshape(BATCH, RETAIN, C),
        z_flat.reshape(BATCH, TOTAL_TOKENS - RETAIN, C),
    )


# speedup vs baseline: 2.9400x; 1.1026x over previous
import functools

import jax
import jax.numpy as jnp
from jax import lax
from jax.experimental import pallas as pl
from jax.experimental.pallas import tpu as pltpu
from jax.experimental.pallas import tpu_sc as plsc

TOTAL_TOKENS = 1024
RETAIN = 256
BATCH = 64
C = 768

ROWS = BATCH * TOTAL_TOKENS
ROWS_Y = BATCH * RETAIN
NW = 32
RPW = ROWS // NW
Y_WORKERS = ROWS_Y // RPW
CHUNK = 64
NCH = RPW // CHUNK


def _build_sampler_kernel():
    info = plsc.get_sparse_core_info()
    nc = info.num_cores
    mesh = plsc.VectorSubcoreMesh(core_axis_name="c", subcore_axis_name="s")

    @functools.partial(
        pl.kernel,
        mesh=mesh,
        out_type=(
            jax.ShapeDtypeStruct((ROWS_Y, C), jnp.float32),
            jax.ShapeDtypeStruct((ROWS - ROWS_Y, C), jnp.float32),
        ),
        scratch_types=[
            pltpu.VMEM((NCH, CHUNK), jnp.int32),
            pltpu.VMEM((CHUNK, C), jnp.float32),
            pltpu.VMEM((CHUNK, C), jnp.float32),
            pltpu.SemaphoreType.DMA,
            pltpu.SemaphoreType.DMA,
            pltpu.SemaphoreType.DMA,
            pltpu.SemaphoreType.DMA,
        ],
    )
    def sampler(x_hbm, idx_hbm, y_hbm, z_hbm, idx_v, buf0, buf1, g0, g1, s0, s1):
        w = lax.axis_index("s") * nc + lax.axis_index("c")
        pltpu.sync_copy(idx_hbm.at[w], idx_v)

        def run(out_ref, obase):
            def gather(c, buf, sem):
                return pltpu.make_async_copy(x_hbm.at[idx_v.at[c]], buf, sem)

            def store(c, buf, sem):
                return pltpu.make_async_copy(
                    buf, out_ref.at[pl.ds(obase + c * CHUNK, CHUNK)], sem
                )

            gather(0, buf0, g0).start()

            def body(i, carry):
                c0 = 2 * i
                gather(c0, buf0, g0).wait()
                store(c0, buf0, s0).start()

                @pl.when(i > 0)
                def _():
                    store(c0 - 1, buf1, s1).wait()

                gather(c0 + 1, buf1, g1).start()
                gather(c0 + 1, buf1, g1).wait()
                store(c0 + 1, buf1, s1).start()
                store(c0, buf0, s0).wait()

                @pl.when(i < NCH // 2 - 1)
                def _():
                    gather(c0 + 2, buf0, g0).start()

                return carry

            lax.fori_loop(0, NCH // 2, body, 0)
            store(NCH - 1, buf1, s1).wait()

        @pl.when(w < Y_WORKERS)
        def _():
            run(y_hbm, w * RPW)

        @pl.when(w >= Y_WORKERS)
        def _():
            run(z_hbm, (w - Y_WORKERS) * RPW)

    return sampler


_sampler = _build_sampler_kernel()


def kernel(x):
    perm = jax.random.permutation(jax.random.key(1), TOTAL_TOKENS)
    row_base = (jnp.arange(BATCH, dtype=jnp.int32) * TOTAL_TOKENS)[:, None]
    idx_y = (row_base + perm[None, :RETAIN]).reshape(-1)
    idx_z = (row_base + perm[None, RETAIN:]).reshape(-1)
    idx = (
        jnp.concatenate([idx_y, idx_z])
        .astype(jnp.int32)
        .reshape(NW, NCH, CHUNK)
    )
    y_flat, z_flat = _sampler(x.reshape(ROWS, C), idx)
    return (
        y_flat.reshape(BATCH, RETAIN, C),
        z_flat.reshape(BATCH, TOTAL_TOKENS - RETAIN, C),
    )
